# Initial kernel scaffold; baseline (speedup 1.0000x reference)
import jax
import jax.numpy as jnp
from jax.experimental import pallas as pl


def kernel(output, target):
    raise NotImplementedError("write your pallas kernel here")


# speedup vs baseline: 4.2889x; 4.2889x over previous
import jax
import jax.numpy as jnp
from jax.experimental import pallas as pl
from jax.experimental.pallas import tpu as pltpu

_BINS = 64
_N = 16777216
_LANES = 128
_ROWS = _N // _LANES
_BLK_ROWS = 4096
_GRID = _ROWS // _BLK_ROWS


def _minmax_body(o_ref, t_ref, mn_ref, mx_ref):
    i = pl.program_id(0)
    bmn = jnp.minimum(jnp.min(o_ref[...]), jnp.min(t_ref[...]))
    bmx = jnp.maximum(jnp.max(o_ref[...]), jnp.max(t_ref[...]))

    @pl.when(i == 0)
    def _():
        mn_ref[0, 0] = bmn
        mx_ref[0, 0] = bmx

    @pl.when(i != 0)
    def _():
        mn_ref[0, 0] = jnp.minimum(mn_ref[0, 0], bmn)
        mx_ref[0, 0] = jnp.maximum(mx_ref[0, 0], bmx)


def _hist_body(mn_ref, mx_ref, o_ref, t_ref, loss_ref, acc_ref):
    i = pl.program_id(0)

    @pl.when(i == 0)
    def _():
        acc_ref[...] = jnp.zeros_like(acc_ref)

    mn = mn_ref[0, 0]
    mx = mx_ref[0, 0]
    scale = _BINS / (mx - mn)
    idx_o = jnp.floor((o_ref[...] - mn) * scale).astype(jnp.int32)
    idx_o = jnp.clip(idx_o, 0, _BINS - 1)
    idx_t = jnp.floor((t_ref[...] - mn) * scale).astype(jnp.int32)
    idx_t = jnp.clip(idx_t, 0, _BINS - 1)

    for b in range(_BINS):
        so = jnp.sum((idx_o == b).astype(jnp.float32), axis=0, keepdims=True)
        st = jnp.sum((idx_t == b).astype(jnp.float32), axis=0, keepdims=True)
        acc_ref[b:b + 1, :] += so
        acc_ref[_BINS + b:_BINS + b + 1, :] += st

    @pl.when(i == _GRID - 1)
    def _():
        ho = jnp.sum(acc_ref[0:_BINS, :], axis=1)
        ht = jnp.sum(acc_ref[_BINS:2 * _BINS, :], axis=1)
        loss_ref[0, 0] = jnp.mean(jnp.abs(ho - ht))


def kernel(output, target):
    o2 = output.reshape(_ROWS, _LANES)
    t2 = target.reshape(_ROWS, _LANES)

    mn, mx = pl.pallas_call(
        _minmax_body,
        grid=(_GRID,),
        in_specs=[
            pl.BlockSpec((_BLK_ROWS, _LANES), lambda i: (i, 0)),
            pl.BlockSpec((_BLK_ROWS, _LANES), lambda i: (i, 0)),
        ],
        out_specs=[
            pl.BlockSpec((1, 1), lambda i: (0, 0), memory_space=pltpu.SMEM),
            pl.BlockSpec((1, 1), lambda i: (0, 0), memory_space=pltpu.SMEM),
        ],
        out_shape=[
            jax.ShapeDtypeStruct((1, 1), jnp.float32),
            jax.ShapeDtypeStruct((1, 1), jnp.float32),
        ],
        compiler_params=pltpu.CompilerParams(
            dimension_semantics=("arbitrary",),
        ),
    )(o2, t2)

    loss = pl.pallas_call(
        _hist_body,
        grid=(_GRID,),
        in_specs=[
            pl.BlockSpec((1, 1), lambda i: (0, 0), memory_space=pltpu.SMEM),
            pl.BlockSpec((1, 1), lambda i: (0, 0), memory_space=pltpu.SMEM),
            pl.BlockSpec((_BLK_ROWS, _LANES), lambda i: (i, 0)),
            pl.BlockSpec((_BLK_ROWS, _LANES), lambda i: (i, 0)),
        ],
        out_specs=pl.BlockSpec((1, 1), lambda i: (0, 0),
                               memory_space=pltpu.SMEM),
        out_shape=jax.ShapeDtypeStruct((1, 1), jnp.float32),
        scratch_shapes=[pltpu.VMEM((2 * _BINS, _LANES), jnp.float32)],
        compiler_params=pltpu.CompilerParams(
            dimension_semantics=("arbitrary",),
        ),
    )(mn, mx, o2, t2)

    return loss[0, 0]


# speedup vs baseline: 4.5356x; 1.0575x over previous
import dataclasses

import jax
import jax.numpy as jnp
from jax import lax
from jax.experimental import pallas as pl
from jax.experimental.pallas import tpu as pltpu
from jax.experimental.pallas import tpu_sc as plsc

_BINS = 64
_N = 16777216
_LANES = 128
_ROWS = _N // _LANES
_BLK_ROWS = 4096
_GRID = _ROWS // _BLK_ROWS

_NC, _NS, _L = 2, 16, 16
_NW = _NC * _NS
_TILE_N = _N // _NW
_CHUNK = 16384
_NCHUNK = _TILE_N // _CHUNK
_UNROLL = 8


def _minmax_body(o_ref, t_ref, ms_ref, mn_sm, mx_sm):
    i = pl.program_id(0)
    bmn = jnp.minimum(jnp.min(o_ref[...]), jnp.min(t_ref[...]))
    bmx = jnp.maximum(jnp.max(o_ref[...]), jnp.max(t_ref[...]))

    @pl.when(i == 0)
    def _():
        mn_sm[0] = bmn
        mx_sm[0] = bmx

    @pl.when(i != 0)
    def _():
        mn_sm[0] = jnp.minimum(mn_sm[0], bmn)
        mx_sm[0] = jnp.maximum(mx_sm[0], bmx)

    @pl.when(i == _GRID - 1)
    def _():
        mn = mn_sm[0]
        scale = _BINS / (mx_sm[0] - mn)
        ms_ref[0:1, :] = jnp.full((1, _LANES), mn, jnp.float32)
        ms_ref[1:2, :] = jnp.full((1, _LANES), scale, jnp.float32)


def _sc_hist_body(o_hbm, t_hbm, ms_hbm, out_hbm, mn_v, sc_v, ob0, ob1, tb0,
                  tb1, ho, ht, sems):
    cid = lax.axis_index("c")
    sid = lax.axis_index("s")
    wid = sid * _NC + cid
    base = wid * _TILE_N

    pltpu.sync_copy(ms_hbm.at[0], mn_v)
    pltpu.sync_copy(ms_hbm.at[1], sc_v)
    mn = mn_v[pl.ds(0, _L)]
    sc = sc_v[pl.ds(0, _L)]

    zeros = jnp.zeros((_L,), jnp.float32)
    for k in range(_BINS // _L):
        ho[pl.ds(k * _L, _L)] = zeros
        ht[pl.ds(k * _L, _L)] = zeros

    obufs = (ob0, ob1)
    tbufs = (tb0, tb1)

    def start(slot, c):
        off = base + c * _CHUNK
        pltpu.async_copy(o_hbm.at[pl.ds(off, _CHUNK)], obufs[slot],
                         sems.at[0, slot])
        pltpu.async_copy(t_hbm.at[pl.ds(off, _CHUNK)], tbufs[slot],
                         sems.at[1, slot])

    def wait(slot):
        pltpu.make_async_copy(o_hbm.at[pl.ds(0, _CHUNK)], obufs[slot],
                              sems.at[0, slot]).wait()
        pltpu.make_async_copy(t_hbm.at[pl.ds(0, _CHUNK)], tbufs[slot],
                              sems.at[1, slot]).wait()

    start(0, 0)
    start(1, 1)

    ones = jnp.full((_L,), 1.0, jnp.float32)
    zf = jnp.zeros((_L,), jnp.float32)
    topf = jnp.full((_L,), float(_BINS - 1), jnp.float32)

    def process(buf, hist):
        @pl.loop(0, _CHUNK, step=_L * _UNROLL)
        def _(j):
            for u in range(_UNROLL):
                x = buf[pl.ds(j + u * _L, _L)]
                t = (x - mn) * sc
                t = jnp.minimum(jnp.maximum(t, zf), topf)
                idx = t.astype(jnp.int32)
                plsc.addupdate_scatter(hist, [idx], ones)

    @pl.loop(0, _NCHUNK, step=2)
    def _(c):
        for b in range(2):
            wait(b)
            process(obufs[b], ho)
            process(tbufs[b], ht)

            @pl.when(c + (b + 2) < _NCHUNK)
            def _():
                start(b, c + (b + 2))

    pltpu.sync_copy(ho, out_hbm.at[wid, pl.ds(0, _BINS)])
    pltpu.sync_copy(ht, out_hbm.at[wid, pl.ds(_BINS, _BINS)])


def _loss_body(ho_ref, ht_ref, loss_ref):
    d = jnp.sum(ho_ref[...], axis=0) - jnp.sum(ht_ref[...], axis=0)
    loss_ref[0, 0] = jnp.mean(jnp.abs(d))


def kernel(output, target):
    o2 = output.reshape(_ROWS, _LANES)
    t2 = target.reshape(_ROWS, _LANES)

    ms = pl.pallas_call(
        _minmax_body,
        grid=(_GRID,),
        in_specs=[
            pl.BlockSpec((_BLK_ROWS, _LANES), lambda i: (i, 0)),
            pl.BlockSpec((_BLK_ROWS, _LANES), lambda i: (i, 0)),
        ],
        out_specs=pl.BlockSpec((2, _LANES), lambda i: (0, 0)),
        out_shape=jax.ShapeDtypeStruct((2, _LANES), jnp.float32),
        scratch_shapes=[
            pltpu.SMEM((1,), jnp.float32),
            pltpu.SMEM((1,), jnp.float32),
        ],
        compiler_params=pltpu.CompilerParams(
            dimension_semantics=("arbitrary",),
        ),
    )(o2, t2)

    sc_params = pltpu.CompilerParams()
    if "needs_layout_passes" in pltpu.CompilerParams.__dataclass_fields__:
        sc_params = dataclasses.replace(sc_params, needs_layout_passes=False)

    sc_hist = pl.kernel(
        _sc_hist_body,
        compiler_params=sc_params,
        out_type=jax.ShapeDtypeStruct((_NW, 2 * _BINS), jnp.float32),
        mesh=plsc.VectorSubcoreMesh(core_axis_name="c", subcore_axis_name="s",
                                    num_cores=_NC, num_subcores=_NS),
        scratch_types=[
            pltpu.VMEM((_LANES,), jnp.float32),
            pltpu.VMEM((_LANES,), jnp.float32),
            pltpu.VMEM((_CHUNK,), jnp.float32),
            pltpu.VMEM((_CHUNK,), jnp.float32),
            pltpu.VMEM((_CHUNK,), jnp.float32),
            pltpu.VMEM((_CHUNK,), jnp.float32),
            pltpu.VMEM((_BINS,), jnp.float32),
            pltpu.VMEM((_BINS,), jnp.float32),
            pltpu.SemaphoreType.DMA((2, 2)),
        ],
    )
    hp = sc_hist(output, target, ms)

    loss = pl.pallas_call(
        _loss_body,
        out_specs=pl.BlockSpec(memory_space=pltpu.SMEM),
        out_shape=jax.ShapeDtypeStruct((1, 1), jnp.float32),
    )(hp[:, :_BINS], hp[:, _BINS:])

    return loss[0, 0]


# speedup vs baseline: 4.5506x; 1.0033x over previous
import dataclasses

import jax
import jax.numpy as jnp
from jax import lax
from jax.experimental import pallas as pl
from jax.experimental.pallas import tpu as pltpu
from jax.experimental.pallas import tpu_sc as plsc

_BINS = 64
_N = 16777216
_LANES = 128
_ROWS = _N // _LANES
_BLK_ROWS = 4096
_GRID = _ROWS // _BLK_ROWS

_NC, _NS, _L = 2, 16, 16
_NW = _NC * _NS
_TILE_N = _N // _NW
_CHUNK = 16384
_NCHUNK = _TILE_N // _CHUNK
_UNROLL = 8


def _minmax_body(o_ref, t_ref, ms_ref, mn_sm, mx_sm):
    i = pl.program_id(0)
    bmn = jnp.minimum(jnp.min(o_ref[...]), jnp.min(t_ref[...]))
    bmx = jnp.maximum(jnp.max(o_ref[...]), jnp.max(t_ref[...]))

    @pl.when(i == 0)
    def _():
        mn_sm[0] = bmn
        mx_sm[0] = bmx

    @pl.when(i != 0)
    def _():
        mn_sm[0] = jnp.minimum(mn_sm[0], bmn)
        mx_sm[0] = jnp.maximum(mx_sm[0], bmx)

    @pl.when(i == _GRID - 1)
    def _():
        mn = mn_sm[0]
        scale = _BINS / (mx_sm[0] - mn)
        ms_ref[0:1, :] = jnp.full((1, _LANES), mn, jnp.float32)
        ms_ref[1:2, :] = jnp.full((1, _LANES), scale, jnp.float32)


def _sc_hist_body(o_hbm, t_hbm, ms_hbm, out_hbm, mn_v, sc_v, ob0, ob1, tb0,
                  tb1, ho0, ho1, ho2, ho3, ht0, ht1, ht2, ht3, sems):
    cid = lax.axis_index("c")
    sid = lax.axis_index("s")
    wid = sid * _NC + cid
    base = wid * _TILE_N

    pltpu.sync_copy(ms_hbm.at[0], mn_v)
    pltpu.sync_copy(ms_hbm.at[1], sc_v)
    mn = mn_v[pl.ds(0, _L)]
    sc = sc_v[pl.ds(0, _L)]

    hos = (ho0, ho1, ho2, ho3)
    hts = (ht0, ht1, ht2, ht3)
    zeros = jnp.zeros((_L,), jnp.float32)
    for h in hos + hts:
        for k in range(_BINS // _L):
            h[pl.ds(k * _L, _L)] = zeros

    obufs = (ob0, ob1)
    tbufs = (tb0, tb1)

    def start(slot, c):
        off = base + c * _CHUNK
        pltpu.async_copy(o_hbm.at[pl.ds(off, _CHUNK)], obufs[slot],
                         sems.at[0, slot])
        pltpu.async_copy(t_hbm.at[pl.ds(off, _CHUNK)], tbufs[slot],
                         sems.at[1, slot])

    def wait(slot):
        pltpu.make_async_copy(o_hbm.at[pl.ds(0, _CHUNK)], obufs[slot],
                              sems.at[0, slot]).wait()
        pltpu.make_async_copy(t_hbm.at[pl.ds(0, _CHUNK)], tbufs[slot],
                              sems.at[1, slot]).wait()

    start(0, 0)
    start(1, 1)

    ones = jnp.full((_L,), 1.0, jnp.float32)
    zf = jnp.zeros((_L,), jnp.float32)
    topf = jnp.full((_L,), float(_BINS - 1), jnp.float32)

    def binvec(x):
        t = (x - mn) * sc
        t = jnp.minimum(jnp.maximum(t, zf), topf)
        return t.astype(jnp.int32)

    def process(obuf, tbuf):
        @pl.loop(0, _CHUNK, step=_L * _UNROLL)
        def _(j):
            for u in range(_UNROLL):
                plsc.addupdate_scatter(
                    hos[u % 4], [binvec(obuf[pl.ds(j + u * _L, _L)])], ones)
                plsc.addupdate_scatter(
                    hts[u % 4], [binvec(tbuf[pl.ds(j + u * _L, _L)])], ones)

    @pl.loop(0, _NCHUNK, step=2)
    def _(c):
        for b in range(2):
            wait(b)
            process(obufs[b], tbufs[b])

            @pl.when(c + (b + 2) < _NCHUNK)
            def _():
                start(b, c + (b + 2))

    for k in range(_BINS // _L):
        s = pl.ds(k * _L, _L)
        ho0[s] = ho0[s] + ho1[s] + ho2[s] + ho3[s]
        ht0[s] = ht0[s] + ht1[s] + ht2[s] + ht3[s]

    pltpu.sync_copy(ho0, out_hbm.at[wid, pl.ds(0, _BINS)])
    pltpu.sync_copy(ht0, out_hbm.at[wid, pl.ds(_BINS, _BINS)])


def _loss_body(ho_ref, ht_ref, loss_ref):
    d = jnp.sum(ho_ref[...], axis=0) - jnp.sum(ht_ref[...], axis=0)
    loss_ref[0, 0] = jnp.mean(jnp.abs(d))


def kernel(output, target):
    o2 = output.reshape(_ROWS, _LANES)
    t2 = target.reshape(_ROWS, _LANES)

    ms = pl.pallas_call(
        _minmax_body,
        grid=(_GRID,),
        in_specs=[
            pl.BlockSpec((_BLK_ROWS, _LANES), lambda i: (i, 0)),
            pl.BlockSpec((_BLK_ROWS, _LANES), lambda i: (i, 0)),
        ],
        out_specs=pl.BlockSpec((2, _LANES), lambda i: (0, 0)),
        out_shape=jax.ShapeDtypeStruct((2, _LANES), jnp.float32),
        scratch_shapes=[
            pltpu.SMEM((1,), jnp.float32),
            pltpu.SMEM((1,), jnp.float32),
        ],
        compiler_params=pltpu.CompilerParams(
            dimension_semantics=("arbitrary",),
        ),
    )(o2, t2)

    sc_params = pltpu.CompilerParams()
    if "needs_layout_passes" in pltpu.CompilerParams.__dataclass_fields__:
        sc_params = dataclasses.replace(sc_params, needs_layout_passes=False)

    sc_hist = pl.kernel(
        _sc_hist_body,
        compiler_params=sc_params,
        out_type=jax.ShapeDtypeStruct((_NW, 2 * _BINS), jnp.float32),
        mesh=plsc.VectorSubcoreMesh(core_axis_name="c", subcore_axis_name="s",
                                    num_cores=_NC, num_subcores=_NS),
        scratch_types=[
            pltpu.VMEM((_LANES,), jnp.float32),
            pltpu.VMEM((_LANES,), jnp.float32),
            pltpu.VMEM((_CHUNK,), jnp.float32),
            pltpu.VMEM((_CHUNK,), jnp.float32),
            pltpu.VMEM((_CHUNK,), jnp.float32),
            pltpu.VMEM((_CHUNK,), jnp.float32),
            pltpu.VMEM((_BINS,), jnp.float32),
            pltpu.VMEM((_BINS,), jnp.float32),
            pltpu.VMEM((_BINS,), jnp.float32),
            pltpu.VMEM((_BINS,), jnp.float32),
            pltpu.VMEM((_BINS,), jnp.float32),
            pltpu.VMEM((_BINS,), jnp.float32),
            pltpu.VMEM((_BINS,), jnp.float32),
            pltpu.VMEM((_BINS,), jnp.float32),
            pltpu.SemaphoreType.DMA((2, 2)),
        ],
    )
    hp = sc_hist(output, target, ms)

    loss = pl.pallas_call(
        _loss_body,
        out_specs=pl.BlockSpec(memory_space=pltpu.SMEM),
        out_shape=jax.ShapeDtypeStruct((1, 1), jnp.float32),
    )(hp[:, :_BINS], hp[:, _BINS:])

    return loss[0, 0]


# speedup vs baseline: 16.6772x; 3.6648x over previous
import dataclasses

import jax
import jax.numpy as jnp
from jax import lax
from jax.experimental import pallas as pl
from jax.experimental.pallas import tpu as pltpu
from jax.experimental.pallas import tpu_sc as plsc

_BINS = 64
_N = 16777216
_LANES = 128
_ROWS = _N // _LANES
_BLK_ROWS = 4096
_GRID = _ROWS // _BLK_ROWS

_NC, _NS, _L = 2, 16, 16
_NW = _NC * _NS
_TILE_N = _N // _NW
_CHUNK = 16384
_NCHUNK = _TILE_N // _CHUNK
_UNROLL = 8


def _minmax_body(o_ref, t_ref, ms_ref, mn_sm, mx_sm):
    i = pl.program_id(0)
    bmn = jnp.minimum(jnp.min(o_ref[...]), jnp.min(t_ref[...]))
    bmx = jnp.maximum(jnp.max(o_ref[...]), jnp.max(t_ref[...]))

    @pl.when(i == 0)
    def _():
        mn_sm[0] = bmn
        mx_sm[0] = bmx

    @pl.when(i != 0)
    def _():
        mn_sm[0] = jnp.minimum(mn_sm[0], bmn)
        mx_sm[0] = jnp.maximum(mx_sm[0], bmx)

    @pl.when(i == _GRID - 1)
    def _():
        mn = mn_sm[0]
        scale = _BINS / (mx_sm[0] - mn)
        ms_ref[0:1, :] = jnp.full((1, _LANES), mn, jnp.float32)
        ms_ref[1:2, :] = jnp.full((1, _LANES), scale, jnp.float32)


def _sc_hist_body(o_hbm, t_hbm, ms_hbm, out_hbm, mn_v, sc_v, ob0, ob1, tb0,
                  tb1, ho0, ho1, ho2, ho3, ht0, ht1, ht2, ht3, sems):
    cid = lax.axis_index("c")
    sid = lax.axis_index("s")
    wid = sid * _NC + cid
    base = wid * _TILE_N

    pltpu.sync_copy(ms_hbm.at[0], mn_v)
    pltpu.sync_copy(ms_hbm.at[1], sc_v)
    mn = mn_v[pl.ds(0, _L)]
    sc = sc_v[pl.ds(0, _L)]

    hos = (ho0, ho1, ho2, ho3)
    hts = (ht0, ht1, ht2, ht3)
    zeros = jnp.zeros((_L,), jnp.float32)
    for h in hos + hts:
        for k in range(_BINS // _L):
            h[pl.ds(k * _L, _L)] = zeros

    obufs = (ob0, ob1)
    tbufs = (tb0, tb1)

    def start(slot, c):
        off = base + c * _CHUNK
        pltpu.async_copy(o_hbm.at[pl.ds(off, _CHUNK)], obufs[slot],
                         sems.at[0, slot])
        pltpu.async_copy(t_hbm.at[pl.ds(off, _CHUNK)], tbufs[slot],
                         sems.at[1, slot])

    def wait(slot):
        pltpu.make_async_copy(o_hbm.at[pl.ds(0, _CHUNK)], obufs[slot],
                              sems.at[0, slot]).wait()
        pltpu.make_async_copy(t_hbm.at[pl.ds(0, _CHUNK)], tbufs[slot],
                              sems.at[1, slot]).wait()

    start(0, 0)
    start(1, 1)

    ones = jnp.full((_L,), 1.0, jnp.float32)
    zf = jnp.zeros((_L,), jnp.float32)
    topf = jnp.full((_L,), float(_BINS - 1), jnp.float32)

    def binvec(x):
        t = (x - mn) * sc
        t = jnp.minimum(jnp.maximum(t, zf), topf)
        return t.astype(jnp.int32)

    def process(obuf, tbuf):
        @pl.loop(0, _CHUNK, step=_L * _UNROLL)
        def _(j):
            xs = []
            for u in range(_UNROLL):
                xs.append(obuf[pl.ds(j + u * _L, _L)])
                xs.append(tbuf[pl.ds(j + u * _L, _L)])
            idxs = [binvec(x) for x in xs]
            for u in range(_UNROLL):
                plsc.addupdate_scatter(hos[u % 4], [idxs[2 * u]], ones)
                plsc.addupdate_scatter(hts[u % 4], [idxs[2 * u + 1]], ones)

    @pl.loop(0, _NCHUNK, step=2)
    def _(c):
        for b in range(2):
            wait(b)
            process(obufs[b], tbufs[b])

            @pl.when(c + (b + 2) < _NCHUNK)
            def _():
                start(b, c + (b + 2))

    for k in range(_BINS // _L):
        s = pl.ds(k * _L, _L)
        ho0[s] = ho0[s] + ho1[s] + ho2[s] + ho3[s]
        ht0[s] = ht0[s] + ht1[s] + ht2[s] + ht3[s]

    pltpu.sync_copy(ho0, out_hbm.at[wid, pl.ds(0, _BINS)])
    pltpu.sync_copy(ht0, out_hbm.at[wid, pl.ds(_BINS, _BINS)])


def _loss_body(ho_ref, ht_ref, loss_ref):
    d = jnp.sum(ho_ref[...], axis=0) - jnp.sum(ht_ref[...], axis=0)
    loss_ref[0, 0] = jnp.mean(jnp.abs(d))


def kernel(output, target):
    o2 = output.reshape(_ROWS, _LANES)
    t2 = target.reshape(_ROWS, _LANES)

    ms = pl.pallas_call(
        _minmax_body,
        grid=(_GRID,),
        in_specs=[
            pl.BlockSpec((_BLK_ROWS, _LANES), lambda i: (i, 0)),
            pl.BlockSpec((_BLK_ROWS, _LANES), lambda i: (i, 0)),
        ],
        out_specs=pl.BlockSpec((2, _LANES), lambda i: (0, 0)),
        out_shape=jax.ShapeDtypeStruct((2, _LANES), jnp.float32),
        scratch_shapes=[
            pltpu.SMEM((1,), jnp.float32),
            pltpu.SMEM((1,), jnp.float32),
        ],
        compiler_params=pltpu.CompilerParams(
            dimension_semantics=("arbitrary",),
        ),
    )(o2, t2)

    sc_params = pltpu.CompilerParams()
    if "needs_layout_passes" in pltpu.CompilerParams.__dataclass_fields__:
        sc_params = dataclasses.replace(sc_params, needs_layout_passes=False)

    sc_hist = pl.kernel(
        _sc_hist_body,
        compiler_params=sc_params,
        out_type=jax.ShapeDtypeStruct((_NW, 2 * _BINS), jnp.float32),
        mesh=plsc.VectorSubcoreMesh(core_axis_name="c", subcore_axis_name="s",
                                    num_cores=_NC, num_subcores=_NS),
        scratch_types=[
            pltpu.VMEM((_LANES,), jnp.float32),
            pltpu.VMEM((_LANES,), jnp.float32),
            pltpu.VMEM((_CHUNK,), jnp.float32),
            pltpu.VMEM((_CHUNK,), jnp.float32),
            pltpu.VMEM((_CHUNK,), jnp.float32),
            pltpu.VMEM((_CHUNK,), jnp.float32),
            pltpu.VMEM((_BINS,), jnp.float32),
            pltpu.VMEM((_BINS,), jnp.float32),
            pltpu.VMEM((_BINS,), jnp.float32),
            pltpu.VMEM((_BINS,), jnp.float32),
            pltpu.VMEM((_BINS,), jnp.float32),
            pltpu.VMEM((_BINS,), jnp.float32),
            pltpu.VMEM((_BINS,), jnp.float32),
            pltpu.VMEM((_BINS,), jnp.float32),
            pltpu.SemaphoreType.DMA((2, 2)),
        ],
    )
    hp = sc_hist(output, target, ms)

    loss = pl.pallas_call(
        _loss_body,
        out_specs=pl.BlockSpec(memory_space=pltpu.SMEM),
        out_shape=jax.ShapeDtypeStruct((1, 1), jnp.float32),
    )(hp[:, :_BINS], hp[:, _BINS:])

    return loss[0, 0]


# speedup vs baseline: 19.0206x; 1.1405x over previous
import dataclasses

import jax
import jax.numpy as jnp
from jax import lax
from jax.experimental import pallas as pl
from jax.experimental.pallas import tpu as pltpu
from jax.experimental.pallas import tpu_sc as plsc

_BINS = 64
_N = 16777216
_LANES = 128
_ROWS = _N // _LANES
_BLK_ROWS = 4096
_GRID = _ROWS // _BLK_ROWS

_NC, _NS, _L = 2, 16, 16
_NW = _NC * _NS
_TILE_N = _N // _NW
_CHUNK = 16384
_NCHUNK = _TILE_N // _CHUNK
_UNROLL = 8


def _minmax_body(o_ref, t_ref, ms_ref, mn_sm, mx_sm):
    i = pl.program_id(0)
    bmn = jnp.minimum(jnp.min(o_ref[...]), jnp.min(t_ref[...]))
    bmx = jnp.maximum(jnp.max(o_ref[...]), jnp.max(t_ref[...]))

    @pl.when(i == 0)
    def _():
        mn_sm[0] = bmn
        mx_sm[0] = bmx

    @pl.when(i != 0)
    def _():
        mn_sm[0] = jnp.minimum(mn_sm[0], bmn)
        mx_sm[0] = jnp.maximum(mx_sm[0], bmx)

    @pl.when(i == _GRID - 1)
    def _():
        mn = mn_sm[0]
        scale = _BINS / (mx_sm[0] - mn)
        ms_ref[0:1, :] = jnp.full((1, _LANES), mn, jnp.float32)
        ms_ref[1:2, :] = jnp.full((1, _LANES), scale, jnp.float32)


def _sc_hist_body(o_hbm, t_hbm, ms_hbm, out_hbm, mn_v, sc_v, ob0, ob1, tb0,
                  tb1, ho0, ht0, sems):
    cid = lax.axis_index("c")
    sid = lax.axis_index("s")
    wid = sid * _NC + cid
    base = wid * _TILE_N

    pltpu.sync_copy(ms_hbm.at[0], mn_v)
    pltpu.sync_copy(ms_hbm.at[1], sc_v)
    mn = mn_v[pl.ds(0, _L)]
    sc = sc_v[pl.ds(0, _L)]

    zeros = jnp.zeros((_L,), jnp.float32)
    for h in (ho0, ht0):
        for k in range(_BINS // _L):
            h[pl.ds(k * _L, _L)] = zeros

    obufs = (ob0, ob1)
    tbufs = (tb0, tb1)

    def start(slot, c):
        off = base + c * _CHUNK
        pltpu.async_copy(o_hbm.at[pl.ds(off, _CHUNK)], obufs[slot],
                         sems.at[0, slot])
        pltpu.async_copy(t_hbm.at[pl.ds(off, _CHUNK)], tbufs[slot],
                         sems.at[1, slot])

    def wait(slot):
        pltpu.make_async_copy(o_hbm.at[pl.ds(0, _CHUNK)], obufs[slot],
                              sems.at[0, slot]).wait()
        pltpu.make_async_copy(t_hbm.at[pl.ds(0, _CHUNK)], tbufs[slot],
                              sems.at[1, slot]).wait()

    start(0, 0)
    start(1, 1)

    ones = jnp.full((_L,), 1.0, jnp.float32)
    topf = jnp.full((_L,), float(_BINS - 1), jnp.float32)

    def binvec(x):
        t = jnp.minimum((x - mn) * sc, topf)
        return t.astype(jnp.int32)

    def process(obuf, tbuf):
        @plsc.parallel_loop(0, _CHUNK, step=_L, unroll=_UNROLL)
        def _(j):
            plsc.addupdate_scatter(ho0, [binvec(obuf[pl.ds(j, _L)])], ones)
            plsc.addupdate_scatter(ht0, [binvec(tbuf[pl.ds(j, _L)])], ones)

    @pl.loop(0, _NCHUNK, step=2)
    def _(c):
        for b in range(2):
            wait(b)
            process(obufs[b], tbufs[b])

            @pl.when(c + (b + 2) < _NCHUNK)
            def _():
                start(b, c + (b + 2))

    pltpu.sync_copy(ho0, out_hbm.at[wid, pl.ds(0, _BINS)])
    pltpu.sync_copy(ht0, out_hbm.at[wid, pl.ds(_BINS, _BINS)])


def _loss_body(ho_ref, ht_ref, loss_ref):
    d = jnp.sum(ho_ref[...], axis=0) - jnp.sum(ht_ref[...], axis=0)
    loss_ref[0, 0] = jnp.mean(jnp.abs(d))


def kernel(output, target):
    o2 = output.reshape(_ROWS, _LANES)
    t2 = target.reshape(_ROWS, _LANES)

    ms = pl.pallas_call(
        _minmax_body,
        grid=(_GRID,),
        in_specs=[
            pl.BlockSpec((_BLK_ROWS, _LANES), lambda i: (i, 0)),
            pl.BlockSpec((_BLK_ROWS, _LANES), lambda i: (i, 0)),
        ],
        out_specs=pl.BlockSpec((2, _LANES), lambda i: (0, 0)),
        out_shape=jax.ShapeDtypeStruct((2, _LANES), jnp.float32),
        scratch_shapes=[
            pltpu.SMEM((1,), jnp.float32),
            pltpu.SMEM((1,), jnp.float32),
        ],
        compiler_params=pltpu.CompilerParams(
            dimension_semantics=("arbitrary",),
        ),
    )(o2, t2)

    sc_params = pltpu.CompilerParams()
    if "needs_layout_passes" in pltpu.CompilerParams.__dataclass_fields__:
        sc_params = dataclasses.replace(sc_params, needs_layout_passes=False)

    sc_hist = pl.kernel(
        _sc_hist_body,
        compiler_params=sc_params,
        out_type=jax.ShapeDtypeStruct((_NW, 2 * _BINS), jnp.float32),
        mesh=plsc.VectorSubcoreMesh(core_axis_name="c", subcore_axis_name="s",
                                    num_cores=_NC, num_subcores=_NS),
        scratch_types=[
            pltpu.VMEM((_LANES,), jnp.float32),
            pltpu.VMEM((_LANES,), jnp.float32),
            pltpu.VMEM((_CHUNK,), jnp.float32),
            pltpu.VMEM((_CHUNK,), jnp.float32),
            pltpu.VMEM((_CHUNK,), jnp.float32),
            pltpu.VMEM((_CHUNK,), jnp.float32),
            pltpu.VMEM((_BINS,), jnp.float32),
            pltpu.VMEM((_BINS,), jnp.float32),
            pltpu.SemaphoreType.DMA((2, 2)),
        ],
    )
    hp = sc_hist(output, target, ms)

    loss = pl.pallas_call(
        _loss_body,
        out_specs=pl.BlockSpec(memory_space=pltpu.SMEM),
        out_shape=jax.ShapeDtypeStruct((1, 1), jnp.float32),
    )(hp[:, :_BINS], hp[:, _BINS:])

    return loss[0, 0]
